# baseline (speedup 1.0000x reference)
import jax
import jax.numpy as jnp
import numpy as np
from jax.experimental import pallas as pl
from jax.experimental.pallas import tpu as pltpu

K = 8
B, N = 16, 1024


def _batchnorm(x, g, b):
    m = jnp.mean(x, axis=(0, 1), keepdims=True)
    v = jnp.var(x, axis=(0, 1), keepdims=True)
    return (x - m) / jnp.sqrt(v + 1e-5) * g + b


def _mlp_apply(layers, x, norms=None):
    for i, l in enumerate(layers):
        x = x @ l["W"] + l["b"]
        if i < len(layers) - 1:
            if norms is not None:
                x = _batchnorm(x, norms[i]["g"], norms[i]["b"])
            x = jax.nn.relu(x)
    return x


def _pointconv(pc, qry, pts, feats, mask):
    d2 = jnp.sum((qry[:, :, None, :] - pts[:, None, :, :]) ** 2, axis=-1)
    d2 = jnp.where(mask[:, None, :] > 0, d2, 1e10)
    _, idx = jax.lax.top_k(-d2, K)
    gat = jax.vmap(lambda a, i: a[i])
    npos = gat(pts, idx)
    nfeat = gat(feats, idx)
    nmask = gat(mask, idx)
    rel = npos - qry[:, :, None, :]
    w = _mlp_apply(pc["wnet"], rel)
    nfeat = nfeat * nmask[..., None]
    agg = jnp.einsum("bnkc,bnkm->bncm", nfeat, w) / K
    agg = agg.reshape(agg.shape[0], agg.shape[1], -1)
    out = _mlp_apply(pc["final"], agg, pc["fnorm"])
    out = out + feats @ pc["res"]["W"] + pc["res"]["b"]
    return out * mask[..., None]


def _head_kernel(unit_ref, *refs):
    unit = unit_ref[...]
    out = refs[-1]
    x = unit
    for i in range(5):
        W = refs[2 * i][...]
        bb = refs[2 * i + 1][...]
        y = jnp.dot(x.reshape(B * N, x.shape[-1]), W,
                    preferred_element_type=jnp.float32).reshape(B, N, -1) + bb
        if i < 4:
            g = refs[10 + 2 * i][...]
            nb = refs[10 + 2 * i + 1][...]
            m = jnp.mean(y, axis=(0, 1), keepdims=True)
            v = jnp.mean((y - m) ** 2, axis=(0, 1), keepdims=True)
            y = (y - m) / jnp.sqrt(v + 1e-5) * g + nb
            y = jax.nn.relu(y)
            if y.shape[-1] == x.shape[-1]:
                y = y + x
        x = y
    out[...] = x


def _decoders(params, unit):
    heads = [("health", 1), ("shield", 1), ("ori", 7), ("pos", 2)]
    results = []
    for hname, width in heads:
        pn = params[hname]
        args = []
        for l in pn["mlp"]:
            args.append(l["W"])
            args.append(l["b"])
        for nm in pn["norm"]:
            args.append(nm["g"])
            args.append(nm["b"])
        r = pl.pallas_call(
            _head_kernel,
            out_shape=jax.ShapeDtypeStruct((B, N, width), jnp.float32),
        )(unit, *args)
        results.append(r)
    return results


def kernel(pre_frame, mask, params):
    orig = jnp.transpose(pre_frame, (0, 2, 1))
    pts = jnp.transpose(pre_frame[:, 13:15, :], (0, 2, 1))
    in_feats = orig
    pred = None
    for pc in params["pointconvs"]:
        pred = _pointconv(pc, pts, pts, in_feats, mask)
        in_feats = jnp.concatenate([in_feats, pred], axis=-1)
    unit = jnp.concatenate([orig, pred], axis=-1)
    h, s, o, p = _decoders(params, unit)[:4]
    return (
        jnp.squeeze(h, -1),
        jnp.squeeze(s, -1),
        jnp.transpose(o, (0, 2, 1)),
        jnp.transpose(p, (0, 2, 1)),
    )
